# baseline (speedup 1.0000x reference)
import functools

import jax
import jax.numpy as jnp
from jax import lax
from jax.experimental import pallas as pl
from jax.experimental.pallas import tpu as pltpu
from jax.experimental.pallas import tpu_sc as plsc

_NUM_CLASSES = 100000
_FEAT_DIM = 64
_BATCH = 16384

_NC = 2
_NS = 16
_NW = _NC * _NS
_BPW = _BATCH // _NW
_ICHUNK = 128
_NCHUNK = _BPW // _ICHUNK
_L = 16


def _center_loss_body(labels_hbm, feat_hbm, centers_hbm, out_hbm,
                      idx_v, rows_v, feat_v, acc_v, fsem, gsem):
    wid = lax.axis_index("s") * _NC + lax.axis_index("c")
    base = wid * _BPW

    fcopy = pltpu.async_copy(feat_hbm.at[pl.ds(base, _BPW), :], feat_v, fsem)

    pltpu.sync_copy(labels_hbm.at[wid], idx_v)

    gathers = []
    for j in range(_NCHUNK):
        gathers.append(
            pltpu.async_copy(
                centers_hbm.at[idx_v.at[j]],
                rows_v.at[pl.ds(j * _ICHUNK, _ICHUNK), :],
                gsem,
            )
        )
    for g in gathers:
        g.wait()
    fcopy.wait()

    def row_body(i, acc):
        for c in range(_FEAT_DIM // _L):
            d = feat_v[i, pl.ds(c * _L, _L)] - rows_v[i, pl.ds(c * _L, _L)]
            acc = acc + d * d
        return acc

    acc = lax.fori_loop(0, _BPW, row_body,
                        jnp.zeros((_L,), jnp.float32), unroll=4)
    acc_v[...] = acc
    pltpu.sync_copy(acc_v, out_hbm.at[wid])


@jax.jit
def _center_loss_sc(labels2d, features, centers):
    mesh = plsc.VectorSubcoreMesh(core_axis_name="c", subcore_axis_name="s")
    partials = pl.kernel(
        _center_loss_body,
        mesh=mesh,
        compiler_params=pltpu.CompilerParams(use_tc_tiling_on_sc=False),
        out_type=jax.ShapeDtypeStruct((_NW, _L), jnp.float32),
        scratch_types=[
            pltpu.VMEM((_NCHUNK, _ICHUNK), jnp.int32),
            pltpu.VMEM((_BPW, _FEAT_DIM), jnp.float32),
            pltpu.VMEM((_BPW, _FEAT_DIM), jnp.float32),
            pltpu.VMEM((_L,), jnp.float32),
            pltpu.SemaphoreType.DMA,
            pltpu.SemaphoreType.DMA,
        ],
    )(labels2d, features, centers)
    return jnp.sum(partials) * (0.5 / _BATCH)


def kernel(features, labels, centers):
    labels2d = labels.astype(jnp.int32).reshape(_NW, _NCHUNK, _ICHUNK)
    return _center_loss_sc(labels2d, features, centers)
